# Initial kernel scaffold; baseline (speedup 1.0000x reference)
import jax
import jax.numpy as jnp
from jax.experimental import pallas as pl


def kernel(batch_item_index, place_correlation, W1, W2, b2, W3, b3, W4, b4, A, Wsa, bsa):
    raise NotImplementedError("write your pallas kernel here")


# speedup vs baseline: 1.4274x; 1.4274x over previous
import functools

import jax
import jax.numpy as jnp
from jax import lax
from jax.experimental import pallas as pl
from jax.experimental.pallas import tpu as pltpu
from jax.experimental.pallas import tpu_sc as plsc

D_IN = 4096
H1 = 200
H = 50
D_OUT = 4096
DA = 20
BB = 16
LL = 1024
NLANE = 16

TILE = 256
NTILES = D_IN // TILE


def _sc_counts(idx):
  mesh = plsc.VectorSubcoreMesh(core_axis_name="c", subcore_axis_name="s")

  @functools.partial(
      pl.kernel,
      mesh=mesh,
      out_type=jax.ShapeDtypeStruct((BB, D_IN * NLANE), jnp.float32),
      compiler_params=pltpu.CompilerParams(needs_layout_passes=False),
      scratch_types=[
          pltpu.VMEM((LL,), jnp.int32),
          pltpu.VMEM((D_IN * NLANE,), jnp.float32),
      ],
  )
  def hist(idx_hbm, out_hbm, idx_v, hist_v):
    c = lax.axis_index("c")
    s = lax.axis_index("s")
    wid = s * 2 + c

    @pl.when(wid < BB)
    def _():
      pltpu.sync_copy(idx_hbm.at[wid], idx_v)

      zeros16 = jnp.zeros((NLANE,), jnp.float32)

      def zero_body(i, _):
        hist_v[pl.ds(i * NLANE, NLANE)] = zeros16
        return ()

      lax.fori_loop(0, D_IN, zero_body, ())

      col = lax.iota(jnp.int32, NLANE)
      ones16 = jnp.ones((NLANE,), jnp.float32)

      def scat_body(j, _):
        iv = idx_v[pl.ds(j * NLANE, NLANE)] * NLANE + col
        plsc.addupdate_scatter(hist_v, [iv], ones16)
        return ()

      lax.fori_loop(0, LL // NLANE, scat_body, ())

      pltpu.sync_copy(hist_v, out_hbm.at[wid])

  return hist(idx).reshape(BB, D_IN, NLANE)


def _nt(a, b):
  return lax.dot_general(a, b, (((1,), (1,)), ((), ())),
                         preferred_element_type=jnp.float32)


def _nn(a, b):
  return lax.dot_general(a, b, (((1,), (0,)), ((), ())),
                         preferred_element_type=jnp.float32)


def _main_body(cnt_ref, w1t_ref, pc_ref, w4_ref, a_ref, wsa_ref, bsa_ref,
               w2_ref, b2_ref, w3_ref, b3_ref, b4_ref, out_ref,
               acc_ref, numer_ref, denom_ref):
  t = pl.program_id(0)

  c3 = cnt_ref[...]
  c_t = jnp.sum(c3, axis=2)
  w1t = w1t_ref[...]

  s_t = jnp.tanh(_nt(a_ref[...], w1t))
  e_t = jnp.exp(s_t)

  @pl.when(t == 0)
  def _():
    acc_ref[...] = jnp.zeros_like(acc_ref)
    numer_ref[...] = jnp.zeros_like(numer_ref)
    denom_ref[...] = jnp.zeros_like(denom_ref)

  denom_ref[...] += _nt(c_t, e_t)
  r_t = (c_t[:, None, :] * e_t[None, :, :]).reshape(BB * DA, TILE)
  numer_ref[...] += _nn(r_t, w1t)

  gt = _nt(w1t, w4_ref[...])
  q = gt * pc_ref[...]
  acc_ref[...] += _nn(c_t, q)

  @pl.when(t == NTILES - 1)
  def _():
    numer = numer_ref[...].reshape(BB, DA, H1)
    denom = denom_ref[...]
    emb = numer / denom[:, :, None]
    lz = jnp.sum(emb * wsa_ref[...][0][None, :, None], axis=1) + bsa_ref[0, 0]
    z = jnp.tanh(lz)
    z = jnp.tanh(_nt(z, w2_ref[...]) + b2_ref[...])
    dz = jnp.tanh(_nt(z, w3_ref[...]) + b3_ref[...])
    y = _nt(dz, w4_ref[...]) + b4_ref[...] + acc_ref[...]
    out_ref[...] = jax.nn.sigmoid(y)


def _tc_main(counts3, w1t, pc, w4, a, wsa, bsa, w2, b2, w3, b3, b4):
  grid = (NTILES,)
  full = lambda shape: pl.BlockSpec(shape, lambda t: (0,) * len(shape))
  return pl.pallas_call(
      _main_body,
      grid=grid,
      in_specs=[
          pl.BlockSpec((BB, TILE, NLANE), lambda t: (0, t, 0)),
          pl.BlockSpec((TILE, H1), lambda t: (t, 0)),
          pl.BlockSpec((TILE, D_OUT), lambda t: (t, 0)),
          full((D_OUT, H1)),
          full((DA, H1)),
          full((1, DA)),
          full((1, 1)),
          full((H, H1)),
          full((1, H)),
          full((H1, H)),
          full((1, H1)),
          full((1, D_OUT)),
      ],
      out_specs=pl.BlockSpec((BB, D_OUT), lambda t: (0, 0)),
      out_shape=jax.ShapeDtypeStruct((BB, D_OUT), jnp.float32),
      scratch_shapes=[
          pltpu.VMEM((BB, D_OUT), jnp.float32),
          pltpu.VMEM((BB * DA, H1), jnp.float32),
          pltpu.VMEM((BB, DA), jnp.float32),
      ],
  )(counts3, w1t, pc, w4, a, wsa, bsa, w2, b2, w3, b3, b4)


def kernel(batch_item_index, place_correlation, W1, W2, b2, W3, b3, W4, b4,
           A, Wsa, bsa):
  counts3 = _sc_counts(batch_item_index)
  return _tc_main(
      counts3,
      W1.T,
      place_correlation,
      W4,
      A,
      Wsa,
      bsa.reshape(1, 1),
      W2,
      b2.reshape(1, H),
      W3,
      b3.reshape(1, H1),
      b4.reshape(1, D_OUT),
  )


# speedup vs baseline: 4.5260x; 3.1708x over previous
import functools

import jax
import jax.numpy as jnp
from jax import lax
from jax.experimental import pallas as pl
from jax.experimental.pallas import tpu as pltpu
from jax.experimental.pallas import tpu_sc as plsc

D_IN = 4096
H1 = 200
H = 50
D_OUT = 4096
DA = 20
BB = 16
LL = 1024
NLANE = 16

TILE = 256
NTILES = D_IN // TILE


def _sc_counts(idx, zeros_flat):
  mesh = plsc.VectorSubcoreMesh(core_axis_name="c", subcore_axis_name="s")

  @functools.partial(
      pl.kernel,
      mesh=mesh,
      out_type=jax.ShapeDtypeStruct((BB, NLANE * D_IN), jnp.float32),
      compiler_params=pltpu.CompilerParams(needs_layout_passes=False),
      scratch_types=[
          pltpu.VMEM((LL,), jnp.int32),
          pltpu.VMEM((NLANE * D_IN,), jnp.float32),
      ],
  )
  def hist(idx_hbm, zeros_hbm, out_hbm, idx_v, hist_v):
    c = lax.axis_index("c")
    s = lax.axis_index("s")
    wid = s * 2 + c

    @pl.when(wid < BB)
    def _():
      pltpu.sync_copy(idx_hbm.at[wid], idx_v)
      pltpu.sync_copy(zeros_hbm, hist_v)

      col = lax.iota(jnp.int32, NLANE) * D_IN
      ones16 = jnp.ones((NLANE,), jnp.float32)

      def scat_body(j, _):
        iv = idx_v[pl.ds(j * NLANE, NLANE)] + col
        plsc.addupdate_scatter(hist_v, [iv], ones16)
        return ()

      lax.fori_loop(0, LL // NLANE, scat_body, ())

      pltpu.sync_copy(hist_v, out_hbm.at[wid])

  return hist(idx, zeros_flat).reshape(BB * NLANE, D_IN)


def _nt(a, b):
  return lax.dot_general(a, b, (((1,), (1,)), ((), ())),
                         preferred_element_type=jnp.float32)


def _nn(a, b):
  return lax.dot_general(a, b, (((1,), (0,)), ((), ())),
                         preferred_element_type=jnp.float32)


def _onehot(rows, cols, fn, div=1):
  r = lax.broadcasted_iota(jnp.int32, (rows, cols), 0)
  c = lax.broadcasted_iota(jnp.int32, (rows, cols), 1)
  return jnp.where(fn(r) == c // div, 1.0, 0.0).astype(jnp.float32)


def _main_body(cnt_ref, w1t_ref, pc_ref, w4_ref, a_ref, wsa_ref, bsa_ref,
               w2_ref, b2_ref, w3_ref, b3_ref, b4_ref, out_ref,
               acc_ref, numer_ref, denom_ref):
  t = pl.program_id(0)

  c3 = cnt_ref[...]
  fold = _onehot(BB, BB * NLANE, lambda r: r, div=NLANE)
  c_t = _nn(fold, c3)
  w1t = w1t_ref[...]

  s_t = jnp.tanh(_nt(a_ref[...], w1t))
  e_t = jnp.exp(s_t)

  @pl.when(t == 0)
  def _():
    acc_ref[...] = jnp.zeros_like(acc_ref)
    numer_ref[...] = jnp.zeros_like(numer_ref)
    denom_ref[...] = jnp.zeros_like(denom_ref)

  denom_ref[...] += _nt(c_t, e_t)
  ohb = _onehot(BB * DA, BB, lambda r: r // DA)
  oha = _onehot(BB * DA, DA, lambda r: r % DA)
  r_t = _nn(ohb, c_t) * _nn(oha, e_t)
  numer_ref[...] += _nn(r_t, w1t)

  gt = _nt(w1t, w4_ref[...])
  q = gt * pc_ref[...]
  acc_ref[...] += _nn(c_t, q)

  @pl.when(t == NTILES - 1)
  def _():
    numer = numer_ref[...].reshape(BB, DA, H1)
    denom = denom_ref[...]
    emb = numer / denom[:, :, None]
    lz = jnp.sum(emb * wsa_ref[...][0][None, :, None], axis=1) + bsa_ref[0, 0]
    z = jnp.tanh(lz)
    z = jnp.tanh(_nt(z, w2_ref[...]) + b2_ref[...])
    dz = jnp.tanh(_nt(z, w3_ref[...]) + b3_ref[...])
    y = _nt(dz, w4_ref[...]) + b4_ref[...] + acc_ref[...]
    out_ref[...] = jax.nn.sigmoid(y)


def _tc_main(counts2, w1t, pc, w4, a, wsa, bsa, w2, b2, w3, b3, b4):
  grid = (NTILES,)
  full = lambda shape: pl.BlockSpec(shape, lambda t: (0,) * len(shape))
  return pl.pallas_call(
      _main_body,
      grid=grid,
      in_specs=[
          pl.BlockSpec((BB * NLANE, TILE), lambda t: (0, t)),
          pl.BlockSpec((TILE, H1), lambda t: (t, 0)),
          pl.BlockSpec((TILE, D_OUT), lambda t: (t, 0)),
          full((D_OUT, H1)),
          full((DA, H1)),
          full((1, DA)),
          full((1, 1)),
          full((H, H1)),
          full((1, H)),
          full((H1, H)),
          full((1, H1)),
          full((1, D_OUT)),
      ],
      out_specs=pl.BlockSpec((BB, D_OUT), lambda t: (0, 0)),
      out_shape=jax.ShapeDtypeStruct((BB, D_OUT), jnp.float32),
      scratch_shapes=[
          pltpu.VMEM((BB, D_OUT), jnp.float32),
          pltpu.VMEM((BB * DA, H1), jnp.float32),
          pltpu.VMEM((BB, DA), jnp.float32),
      ],
  )(counts2, w1t, pc, w4, a, wsa, bsa, w2, b2, w3, b3, b4)


def kernel(batch_item_index, place_correlation, W1, W2, b2, W3, b3, W4, b4,
           A, Wsa, bsa):
  zeros_flat = jnp.zeros((NLANE * D_IN,), jnp.float32)
  counts2 = _sc_counts(batch_item_index, zeros_flat)
  return _tc_main(
      counts2,
      W1.T,
      place_correlation,
      W4,
      A,
      Wsa,
      bsa.reshape(1, 1),
      W2,
      b2.reshape(1, H),
      W3,
      b3.reshape(1, H1),
      b4.reshape(1, D_OUT),
  )


# speedup vs baseline: 4.8324x; 1.0677x over previous
import functools

import jax
import jax.numpy as jnp
from jax import lax
from jax.experimental import pallas as pl
from jax.experimental.pallas import tpu as pltpu
from jax.experimental.pallas import tpu_sc as plsc

D_IN = 4096
H1 = 200
H = 50
D_OUT = 4096
DA = 20
BB = 16
LL = 1024
NLANE = 16

TILE = 512
NTILES = D_IN // TILE


def _sc_counts(idx, zeros_flat):
  mesh = plsc.VectorSubcoreMesh(core_axis_name="c", subcore_axis_name="s")

  @functools.partial(
      pl.kernel,
      mesh=mesh,
      out_type=jax.ShapeDtypeStruct((BB, NLANE * D_IN), jnp.float32),
      compiler_params=pltpu.CompilerParams(needs_layout_passes=False),
      scratch_types=[
          pltpu.VMEM((LL,), jnp.int32),
          pltpu.VMEM((NLANE * D_IN,), jnp.float32),
      ],
  )
  def hist(idx_hbm, zeros_hbm, out_hbm, idx_v, hist_v):
    c = lax.axis_index("c")
    s = lax.axis_index("s")
    wid = s * 2 + c

    @pl.when(wid < BB)
    def _():
      pltpu.sync_copy(idx_hbm.at[wid], idx_v)
      pltpu.sync_copy(zeros_hbm, hist_v)

      col = lax.iota(jnp.int32, NLANE) * D_IN
      ones16 = jnp.ones((NLANE,), jnp.float32)

      def scat_body(j, _):
        iv = idx_v[pl.ds(j * NLANE, NLANE)] + col
        plsc.addupdate_scatter(hist_v, [iv], ones16)
        return ()

      lax.fori_loop(0, LL // NLANE, scat_body, ())

      pltpu.sync_copy(hist_v, out_hbm.at[wid])

  return hist(idx, zeros_flat).reshape(BB * NLANE, D_IN)


def _nt(a, b):
  return lax.dot_general(a, b, (((1,), (1,)), ((), ())),
                         preferred_element_type=jnp.float32)


def _nn(a, b):
  return lax.dot_general(a, b, (((1,), (0,)), ((), ())),
                         preferred_element_type=jnp.float32)


def _onehot(rows, cols, fn, div=1):
  r = lax.broadcasted_iota(jnp.int32, (rows, cols), 0)
  c = lax.broadcasted_iota(jnp.int32, (rows, cols), 1)
  return jnp.where(fn(r) == c // div, 1.0, 0.0).astype(jnp.float32)


def _main_body(cnt_ref, w1t_ref, pc_ref, w4_ref, a_ref, wsa_ref, bsa_ref,
               w2_ref, b2_ref, w3_ref, b3_ref, b4_ref, out_ref,
               acc_ref, numer_ref, denom_ref):
  t = pl.program_id(0)

  c3 = cnt_ref[...]
  fold = _onehot(BB, BB * NLANE, lambda r: r, div=NLANE)
  c_t = _nn(fold, c3)
  w1t = w1t_ref[...]

  s_t = jnp.tanh(_nt(a_ref[...], w1t))
  e_t = jnp.exp(s_t)

  @pl.when(t == 0)
  def _():
    acc_ref[...] = jnp.zeros_like(acc_ref)
    numer_ref[...] = jnp.zeros_like(numer_ref)
    denom_ref[...] = jnp.zeros_like(denom_ref)

  denom_ref[...] += _nt(c_t, e_t)
  ohb = _onehot(BB * DA, BB, lambda r: r // DA)
  oha = _onehot(BB * DA, DA, lambda r: r % DA)
  r_t = _nn(ohb, c_t) * _nn(oha, e_t)
  numer_ref[...] += _nn(r_t, w1t)

  gt = _nt(w1t, w4_ref[...])
  q = gt * pc_ref[...]
  acc_ref[...] += _nn(c_t, q)

  @pl.when(t == NTILES - 1)
  def _():
    numer = numer_ref[...].reshape(BB, DA, H1)
    denom = denom_ref[...]
    emb = numer / denom[:, :, None]
    lz = jnp.sum(emb * wsa_ref[...][0][None, :, None], axis=1) + bsa_ref[0, 0]
    z = jnp.tanh(lz)
    z = jnp.tanh(_nt(z, w2_ref[...]) + b2_ref[...])
    dz = jnp.tanh(_nt(z, w3_ref[...]) + b3_ref[...])
    y = _nt(dz, w4_ref[...]) + b4_ref[...] + acc_ref[...]
    out_ref[...] = jax.nn.sigmoid(y)


def _tc_main(counts2, w1t, pc, w4, a, wsa, bsa, w2, b2, w3, b3, b4):
  grid = (NTILES,)
  full = lambda shape: pl.BlockSpec(shape, lambda t: (0,) * len(shape))
  return pl.pallas_call(
      _main_body,
      grid=grid,
      in_specs=[
          pl.BlockSpec((BB * NLANE, TILE), lambda t: (0, t)),
          pl.BlockSpec((TILE, H1), lambda t: (t, 0)),
          pl.BlockSpec((TILE, D_OUT), lambda t: (t, 0)),
          full((D_OUT, H1)),
          full((DA, H1)),
          full((1, DA)),
          full((1, 1)),
          full((H, H1)),
          full((1, H)),
          full((H1, H)),
          full((1, H1)),
          full((1, D_OUT)),
      ],
      out_specs=pl.BlockSpec((BB, D_OUT), lambda t: (0, 0)),
      out_shape=jax.ShapeDtypeStruct((BB, D_OUT), jnp.float32),
      scratch_shapes=[
          pltpu.VMEM((BB, D_OUT), jnp.float32),
          pltpu.VMEM((BB * DA, H1), jnp.float32),
          pltpu.VMEM((BB, DA), jnp.float32),
      ],
  )(counts2, w1t, pc, w4, a, wsa, bsa, w2, b2, w3, b3, b4)


def kernel(batch_item_index, place_correlation, W1, W2, b2, W3, b3, W4, b4,
           A, Wsa, bsa):
  zeros_flat = jnp.zeros((NLANE * D_IN,), jnp.float32)
  counts2 = _sc_counts(batch_item_index, zeros_flat)
  return _tc_main(
      counts2,
      W1.T,
      place_correlation,
      W4,
      A,
      Wsa,
      bsa.reshape(1, 1),
      W2,
      b2.reshape(1, H),
      W3,
      b3.reshape(1, H1),
      b4.reshape(1, D_OUT),
  )


# speedup vs baseline: 5.0054x; 1.0358x over previous
import functools

import jax
import jax.numpy as jnp
from jax import lax
from jax.experimental import pallas as pl
from jax.experimental.pallas import tpu as pltpu
from jax.experimental.pallas import tpu_sc as plsc

D_IN = 4096
H1 = 200
H = 50
D_OUT = 4096
DA = 20
BB = 16
LL = 1024
NLANE = 16

TILE = 512
NTILES = D_IN // TILE


def _sc_counts(idx, zeros_flat):
  mesh = plsc.VectorSubcoreMesh(core_axis_name="c", subcore_axis_name="s")

  @functools.partial(
      pl.kernel,
      mesh=mesh,
      out_type=jax.ShapeDtypeStruct((BB, NLANE * D_IN), jnp.float32),
      compiler_params=pltpu.CompilerParams(needs_layout_passes=False),
      scratch_types=[
          pltpu.VMEM((LL,), jnp.int32),
          pltpu.VMEM((NLANE * D_IN,), jnp.float32),
      ],
  )
  def hist(idx_hbm, zeros_hbm, out_hbm, idx_v, hist_v):
    c = lax.axis_index("c")
    s = lax.axis_index("s")
    wid = s * 2 + c

    @pl.when(wid < BB)
    def _():
      pltpu.sync_copy(idx_hbm.at[wid], idx_v)
      pltpu.sync_copy(zeros_hbm, hist_v)

      col = lax.iota(jnp.int32, NLANE) * D_IN
      ones16 = jnp.ones((NLANE,), jnp.float32)

      def scat_body(j, _):
        iv = idx_v[pl.ds(j * NLANE, NLANE)] + col
        plsc.addupdate_scatter(hist_v, [iv], ones16)
        return ()

      lax.fori_loop(0, LL // NLANE, scat_body, ())

      pltpu.sync_copy(hist_v, out_hbm.at[wid])

  return hist(idx, zeros_flat).reshape(BB * NLANE, D_IN)


def _nt(a, b):
  return lax.dot_general(a, b, (((1,), (1,)), ((), ())),
                         preferred_element_type=jnp.float32)


def _nn(a, b):
  return lax.dot_general(a, b, (((1,), (0,)), ((), ())),
                         preferred_element_type=jnp.float32)


def _tn(a, b):
  return lax.dot_general(a, b, (((0,), (1,)), ((), ())),
                         preferred_element_type=jnp.float32)


def _onehot(rows, cols, fn, div=1):
  r = lax.broadcasted_iota(jnp.int32, (rows, cols), 0)
  c = lax.broadcasted_iota(jnp.int32, (rows, cols), 1)
  return jnp.where(fn(r) == c // div, 1.0, 0.0).astype(jnp.float32)


def _main_body(cnt_ref, w1t_ref, pc_ref, w4_ref, a_ref, wsa_ref, bsa_ref,
               w2_ref, b2_ref, w3_ref, b3_ref, b4_ref, out_ref,
               acc_ref, numer_ref, denom_ref):
  t = pl.program_id(0)

  c3 = cnt_ref[...]
  fold = _onehot(BB, BB * NLANE, lambda r: r, div=NLANE)
  c_t = _nn(fold, c3)
  w1 = w1t_ref[...]

  s_t = jnp.tanh(_nn(a_ref[...], w1))
  e_t = jnp.exp(s_t)

  @pl.when(t == 0)
  def _():
    acc_ref[...] = jnp.zeros_like(acc_ref)
    numer_ref[...] = jnp.zeros_like(numer_ref)
    denom_ref[...] = jnp.zeros_like(denom_ref)

  denom_ref[...] += _nt(c_t, e_t)
  ohb = _onehot(BB * DA, BB, lambda r: r // DA)
  oha = _onehot(BB * DA, DA, lambda r: r % DA)
  r_t = _nn(ohb, c_t) * _nn(oha, e_t)
  numer_ref[...] += _nt(r_t, w1)

  gt = _tn(w1, w4_ref[...])
  q = gt * pc_ref[...]
  acc_ref[...] += _nn(c_t, q)

  @pl.when(t == NTILES - 1)
  def _():
    numer = numer_ref[...].reshape(BB, DA, H1)
    denom = denom_ref[...]
    emb = numer / denom[:, :, None]
    lz = jnp.sum(emb * wsa_ref[...][0][None, :, None], axis=1) + bsa_ref[0, 0]
    z = jnp.tanh(lz)
    z = jnp.tanh(_nt(z, w2_ref[...]) + b2_ref[...])
    dz = jnp.tanh(_nt(z, w3_ref[...]) + b3_ref[...])
    y = _nt(dz, w4_ref[...]) + b4_ref[...] + acc_ref[...]
    out_ref[...] = jax.nn.sigmoid(y)


def _tc_main(counts2, w1t, pc, w4, a, wsa, bsa, w2, b2, w3, b3, b4):
  grid = (NTILES,)
  full = lambda shape: pl.BlockSpec(shape, lambda t: (0,) * len(shape))
  return pl.pallas_call(
      _main_body,
      grid=grid,
      in_specs=[
          pl.BlockSpec((BB * NLANE, TILE), lambda t: (0, t)),
          pl.BlockSpec((H1, TILE), lambda t: (0, t)),
          pl.BlockSpec((TILE, D_OUT), lambda t: (t, 0)),
          full((D_OUT, H1)),
          full((DA, H1)),
          full((1, DA)),
          full((1, 1)),
          full((H, H1)),
          full((1, H)),
          full((H1, H)),
          full((1, H1)),
          full((1, D_OUT)),
      ],
      out_specs=pl.BlockSpec((BB, D_OUT), lambda t: (0, 0)),
      out_shape=jax.ShapeDtypeStruct((BB, D_OUT), jnp.float32),
      scratch_shapes=[
          pltpu.VMEM((BB, D_OUT), jnp.float32),
          pltpu.VMEM((BB * DA, H1), jnp.float32),
          pltpu.VMEM((BB, DA), jnp.float32),
      ],
  )(counts2, w1t, pc, w4, a, wsa, bsa, w2, b2, w3, b3, b4)


def kernel(batch_item_index, place_correlation, W1, W2, b2, W3, b3, W4, b4,
           A, Wsa, bsa):
  zeros_flat = jnp.zeros((NLANE * D_IN,), jnp.float32)
  counts2 = _sc_counts(batch_item_index, zeros_flat)
  return _tc_main(
      counts2,
      W1,
      place_correlation,
      W4,
      A,
      Wsa,
      bsa.reshape(1, 1),
      W2,
      b2.reshape(1, H),
      W3,
      b3.reshape(1, H1),
      b4.reshape(1, D_OUT),
  )


# speedup vs baseline: 5.4303x; 1.0849x over previous
import functools

import jax
import jax.numpy as jnp
from jax import lax
from jax.experimental import pallas as pl
from jax.experimental.pallas import tpu as pltpu
from jax.experimental.pallas import tpu_sc as plsc

D_IN = 4096
H1 = 200
H = 50
D_OUT = 4096
DA = 20
BB = 16
LL = 1024
NLANE = 16

TILE = 512
NTILES = D_IN // TILE


def _sc_counts(idx, zeros_flat):
  mesh = plsc.VectorSubcoreMesh(core_axis_name="c", subcore_axis_name="s")

  @functools.partial(
      pl.kernel,
      mesh=mesh,
      out_type=jax.ShapeDtypeStruct((BB * NLANE, D_IN), jnp.float32),
      compiler_params=pltpu.CompilerParams(needs_layout_passes=False),
      scratch_types=[
          pltpu.VMEM((LL,), jnp.int32),
          pltpu.VMEM((NLANE, D_IN), jnp.float32),
      ],
  )
  def hist(idx_hbm, zeros_hbm, out_hbm, idx_v, hist_v):
    c = lax.axis_index("c")
    s = lax.axis_index("s")
    wid = s * 2 + c

    @pl.when(wid < BB)
    def _():
      pltpu.sync_copy(idx_hbm.at[wid], idx_v)
      pltpu.sync_copy(zeros_hbm, hist_v)

      plane = lax.iota(jnp.int32, NLANE)
      ones16 = jnp.ones((NLANE,), jnp.float32)

      def scat_body(j, _):
        iv = idx_v[pl.ds(j * NLANE, NLANE)]
        plsc.addupdate_scatter(hist_v, [plane, iv], ones16)
        return ()

      lax.fori_loop(0, LL // NLANE, scat_body, ())

      pltpu.sync_copy(hist_v, out_hbm.at[pl.ds(wid * NLANE, NLANE)])

  return hist(idx, zeros_flat)


def _nt(a, b):
  return lax.dot_general(a, b, (((1,), (1,)), ((), ())),
                         preferred_element_type=jnp.float32)


def _nn(a, b):
  return lax.dot_general(a, b, (((1,), (0,)), ((), ())),
                         preferred_element_type=jnp.float32)


def _tn(a, b):
  return lax.dot_general(a, b, (((0,), (1,)), ((), ())),
                         preferred_element_type=jnp.float32)


def _onehot(rows, cols, fn, div=1):
  r = lax.broadcasted_iota(jnp.int32, (rows, cols), 0)
  c = lax.broadcasted_iota(jnp.int32, (rows, cols), 1)
  return jnp.where(fn(r) == c // div, 1.0, 0.0).astype(jnp.float32)


def _main_body(cnt_ref, w1t_ref, pc_ref, w4_ref, a_ref, wsa_ref, bsa_ref,
               w2_ref, b2_ref, w3_ref, b3_ref, b4_ref, out_ref,
               acc_ref, numer_ref, denom_ref):
  t = pl.program_id(0)

  c3 = cnt_ref[...]
  fold = _onehot(BB, BB * NLANE, lambda r: r, div=NLANE)
  c_t = _nn(fold, c3)
  w1 = w1t_ref[...]

  s_t = jnp.tanh(_nn(a_ref[...], w1))
  e_t = jnp.exp(s_t)

  @pl.when(t == 0)
  def _():
    acc_ref[...] = jnp.zeros_like(acc_ref)
    numer_ref[...] = jnp.zeros_like(numer_ref)
    denom_ref[...] = jnp.zeros_like(denom_ref)

  denom_ref[...] += _nt(c_t, e_t)
  ohb = _onehot(BB * DA, BB, lambda r: r // DA)
  oha = _onehot(BB * DA, DA, lambda r: r % DA)
  r_t = _nn(ohb, c_t) * _nn(oha, e_t)
  numer_ref[...] += _nt(r_t, w1)

  gt = _tn(w1, w4_ref[...])
  q = gt * pc_ref[...]
  acc_ref[...] += _nn(c_t, q)

  @pl.when(t == NTILES - 1)
  def _():
    numer = numer_ref[...].reshape(BB, DA, H1)
    denom = denom_ref[...]
    emb = numer / denom[:, :, None]
    lz = jnp.sum(emb * wsa_ref[...][0][None, :, None], axis=1) + bsa_ref[0, 0]
    z = jnp.tanh(lz)
    z = jnp.tanh(_nt(z, w2_ref[...]) + b2_ref[...])
    dz = jnp.tanh(_nt(z, w3_ref[...]) + b3_ref[...])
    y = _nt(dz, w4_ref[...]) + b4_ref[...] + acc_ref[...]
    out_ref[...] = jax.nn.sigmoid(y)


def _tc_main(counts2, w1t, pc, w4, a, wsa, bsa, w2, b2, w3, b3, b4):
  grid = (NTILES,)
  full = lambda shape: pl.BlockSpec(shape, lambda t: (0,) * len(shape))
  return pl.pallas_call(
      _main_body,
      grid=grid,
      in_specs=[
          pl.BlockSpec((BB * NLANE, TILE), lambda t: (0, t)),
          pl.BlockSpec((H1, TILE), lambda t: (0, t)),
          pl.BlockSpec((TILE, D_OUT), lambda t: (t, 0)),
          full((D_OUT, H1)),
          full((DA, H1)),
          full((1, DA)),
          full((1, 1)),
          full((H, H1)),
          full((1, H)),
          full((H1, H)),
          full((1, H1)),
          full((1, D_OUT)),
      ],
      out_specs=pl.BlockSpec((BB, D_OUT), lambda t: (0, 0)),
      out_shape=jax.ShapeDtypeStruct((BB, D_OUT), jnp.float32),
      scratch_shapes=[
          pltpu.VMEM((BB, D_OUT), jnp.float32),
          pltpu.VMEM((BB * DA, H1), jnp.float32),
          pltpu.VMEM((BB, DA), jnp.float32),
      ],
  )(counts2, w1t, pc, w4, a, wsa, bsa, w2, b2, w3, b3, b4)


def kernel(batch_item_index, place_correlation, W1, W2, b2, W3, b3, W4, b4,
           A, Wsa, bsa):
  zeros_flat = jnp.zeros((NLANE, D_IN), jnp.float32)
  counts2 = _sc_counts(batch_item_index, zeros_flat)
  return _tc_main(
      counts2,
      W1,
      place_correlation,
      W4,
      A,
      Wsa,
      bsa.reshape(1, 1),
      W2,
      b2.reshape(1, H),
      W3,
      b3.reshape(1, H1),
      b4.reshape(1, D_OUT),
  )


# speedup vs baseline: 5.9519x; 1.0960x over previous
import functools

import jax
import jax.numpy as jnp
from jax import lax
from jax.experimental import pallas as pl
from jax.experimental.pallas import tpu as pltpu
from jax.experimental.pallas import tpu_sc as plsc

D_IN = 4096
H1 = 200
H = 50
D_OUT = 4096
DA = 20
BB = 16
LL = 1024
NLANE = 16
NPLANE = 4

TILE = 512
NTILES = D_IN // TILE


def _sc_counts(idx, zeros_flat):
  mesh = plsc.VectorSubcoreMesh(core_axis_name="c", subcore_axis_name="s")

  @functools.partial(
      pl.kernel,
      mesh=mesh,
      out_type=jax.ShapeDtypeStruct((BB * NPLANE, D_IN), jnp.float32),
      compiler_params=pltpu.CompilerParams(needs_layout_passes=False),
      scratch_types=[
          pltpu.VMEM((LL,), jnp.int32),
          pltpu.VMEM((NPLANE, D_IN), jnp.float32),
          pltpu.SemaphoreType.DMA,
          pltpu.SemaphoreType.DMA,
      ],
  )
  def hist(idx_hbm, zeros_hbm, out_hbm, idx_v, hist_v, sem1, sem2):
    c = lax.axis_index("c")
    s = lax.axis_index("s")
    wid = s * 2 + c

    @pl.when(wid < BB)
    def _():
      cp1 = pltpu.async_copy(idx_hbm.at[wid], idx_v, sem1)
      cp2 = pltpu.async_copy(zeros_hbm, hist_v, sem2)
      cp1.wait()
      cp2.wait()

      lane = lax.iota(jnp.int32, NLANE)
      ones16 = jnp.ones((NLANE,), jnp.float32)
      planes = [(lane - g * NPLANE) & (NPLANE - 1)
                for g in range(NLANE // NPLANE)]
      masks = [(lane >= g * NPLANE) & (lane < (g + 1) * NPLANE)
               for g in range(NLANE // NPLANE)]

      def scat_body(j, _):
        iv = idx_v[pl.ds(j * NLANE, NLANE)]
        for g in range(NLANE // NPLANE):
          plsc.addupdate_scatter(hist_v, [planes[g], iv], ones16,
                                 mask=masks[g])
        return ()

      lax.fori_loop(0, LL // NLANE, scat_body, ())

      pltpu.sync_copy(hist_v, out_hbm.at[pl.ds(wid * NPLANE, NPLANE)])

  return hist(idx, zeros_flat)


def _nt(a, b):
  return lax.dot_general(a, b, (((1,), (1,)), ((), ())),
                         preferred_element_type=jnp.float32)


def _nn(a, b):
  return lax.dot_general(a, b, (((1,), (0,)), ((), ())),
                         preferred_element_type=jnp.float32)


def _tn(a, b):
  return lax.dot_general(a, b, (((0,), (1,)), ((), ())),
                         preferred_element_type=jnp.float32)


def _onehot(rows, cols, fn, div=1):
  r = lax.broadcasted_iota(jnp.int32, (rows, cols), 0)
  c = lax.broadcasted_iota(jnp.int32, (rows, cols), 1)
  return jnp.where(fn(r) == c // div, 1.0, 0.0).astype(jnp.float32)


def _main_body(cnt_ref, w1t_ref, pc_ref, w4_ref, a_ref, wsa_ref, bsa_ref,
               w2_ref, b2_ref, w3_ref, b3_ref, b4_ref, out_ref,
               acc_ref, numer_ref, denom_ref):
  t = pl.program_id(0)

  c3 = cnt_ref[...]
  fold = _onehot(BB, BB * NPLANE, lambda r: r, div=NPLANE)
  c_t = _nn(fold, c3)
  w1 = w1t_ref[...]

  s_t = jnp.tanh(_nn(a_ref[...], w1))
  e_t = jnp.exp(s_t)

  @pl.when(t == 0)
  def _():
    acc_ref[...] = jnp.zeros_like(acc_ref)
    numer_ref[...] = jnp.zeros_like(numer_ref)
    denom_ref[...] = jnp.zeros_like(denom_ref)

  denom_ref[...] += _nt(c_t, e_t)
  ohb = _onehot(BB * DA, BB, lambda r: r // DA)
  oha = _onehot(BB * DA, DA, lambda r: r % DA)
  r_t = _nn(ohb, c_t) * _nn(oha, e_t)
  numer_ref[...] += _nt(r_t, w1)

  gt = _tn(w1, w4_ref[...])
  q = gt * pc_ref[...]
  acc_ref[...] += _nn(c_t, q)

  @pl.when(t == NTILES - 1)
  def _():
    numer = numer_ref[...].reshape(BB, DA, H1)
    denom = denom_ref[...]
    emb = numer / denom[:, :, None]
    lz = jnp.sum(emb * wsa_ref[...][0][None, :, None], axis=1) + bsa_ref[0, 0]
    z = jnp.tanh(lz)
    z = jnp.tanh(_nt(z, w2_ref[...]) + b2_ref[...])
    dz = jnp.tanh(_nt(z, w3_ref[...]) + b3_ref[...])
    y = _nt(dz, w4_ref[...]) + b4_ref[...] + acc_ref[...]
    out_ref[...] = jax.nn.sigmoid(y)


def _tc_main(counts2, w1t, pc, w4, a, wsa, bsa, w2, b2, w3, b3, b4):
  grid = (NTILES,)
  full = lambda shape: pl.BlockSpec(shape, lambda t: (0,) * len(shape))
  return pl.pallas_call(
      _main_body,
      grid=grid,
      in_specs=[
          pl.BlockSpec((BB * NPLANE, TILE), lambda t: (0, t)),
          pl.BlockSpec((H1, TILE), lambda t: (0, t)),
          pl.BlockSpec((TILE, D_OUT), lambda t: (t, 0)),
          full((D_OUT, H1)),
          full((DA, H1)),
          full((1, DA)),
          full((1, 1)),
          full((H, H1)),
          full((1, H)),
          full((H1, H)),
          full((1, H1)),
          full((1, D_OUT)),
      ],
      out_specs=pl.BlockSpec((BB, D_OUT), lambda t: (0, 0)),
      out_shape=jax.ShapeDtypeStruct((BB, D_OUT), jnp.float32),
      scratch_shapes=[
          pltpu.VMEM((BB, D_OUT), jnp.float32),
          pltpu.VMEM((BB * DA, H1), jnp.float32),
          pltpu.VMEM((BB, DA), jnp.float32),
      ],
  )(counts2, w1t, pc, w4, a, wsa, bsa, w2, b2, w3, b3, b4)


def kernel(batch_item_index, place_correlation, W1, W2, b2, W3, b3, W4, b4,
           A, Wsa, bsa):
  zeros_flat = jnp.zeros((NPLANE, D_IN), jnp.float32)
  counts2 = _sc_counts(batch_item_index, zeros_flat)
  return _tc_main(
      counts2,
      W1,
      place_correlation,
      W4,
      A,
      Wsa,
      bsa.reshape(1, 1),
      W2,
      b2.reshape(1, H),
      W3,
      b3.reshape(1, H1),
      b4.reshape(1, D_OUT),
  )
